# Initial kernel scaffold; baseline (speedup 1.0000x reference)
import jax
import jax.numpy as jnp
from jax.experimental import pallas as pl


def kernel(x, edge_index, batch, W_embed, b_embed, W1, b1, W2, b2, Wt1, bt1, gamma, beta, Wt2, bt2):
    raise NotImplementedError("write your pallas kernel here")


# speedup vs baseline: 6.9732x; 6.9732x over previous
import functools

import jax
import jax.numpy as jnp
from jax import lax
from jax.experimental import pallas as pl
from jax.experimental.pallas import tpu as pltpu
from jax.experimental.pallas import tpu_sc as plsc

N = 10000
E = 320000
D = 128
G = 256
NC = 2
NS = 16
NW = NC * NS

NPAD = 10240
ROWS_PER_TILE = NPAD // NW
ROWS_PER_SUB = NPAD // NS
EC = 10240
CH = 64
NCH = EC // CH
GPAD = 384
DUMMY = N


_MESH = plsc.VectorSubcoreMesh(core_axis_name="c", subcore_axis_name="s")


NCHH = NCH // 2


def _hop_body(g_hbm, srcp, dstp, z_hbm, out_hbm,
              sidx, didx, rows0, rows1, acc, sem0, sem1):
    c = lax.axis_index("c")
    s = lax.axis_index("s")
    w = c * NS + s
    pltpu.sync_copy(srcp.at[w, pl.ds(0, NCHH)], sidx)
    pltpu.sync_copy(dstp.at[w, pl.ds(0, NCHH)], didx)
    pltpu.async_copy(g_hbm.at[sidx.at[0]], rows0, sem0)
    pltpu.sync_copy(z_hbm, acc.at[pl.ds(s * ROWS_PER_SUB, ROWS_PER_SUB)])
    plsc.subcore_barrier()

    def body(j2, carry):
        j = j2 * 2
        pltpu.async_copy(g_hbm.at[sidx.at[j + 1]], rows1, sem1)
        pltpu.make_async_copy(g_hbm.at[sidx.at[j]], rows0, sem0).wait()
        pltpu.sync_copy(rows0, acc.at[didx.at[j]], add=True)

        @pl.when(j2 < NCHH // 2 - 1)
        def _():
            pltpu.async_copy(g_hbm.at[sidx.at[j + 2]], rows0, sem0)

        pltpu.make_async_copy(g_hbm.at[sidx.at[j + 1]], rows1, sem1).wait()
        pltpu.sync_copy(rows1, acc.at[didx.at[j + 1]], add=True)
        return carry

    lax.fori_loop(0, NCHH // 2, body, 0)
    pltpu.sync_copy(srcp.at[w, pl.ds(NCHH, NCHH)], sidx)
    pltpu.sync_copy(dstp.at[w, pl.ds(NCHH, NCHH)], didx)
    pltpu.async_copy(g_hbm.at[sidx.at[0]], rows0, sem0)
    lax.fori_loop(0, NCHH // 2, body, 0)
    plsc.subcore_barrier()
    pltpu.sync_copy(acc.at[pl.ds(s * ROWS_PER_SUB, ROWS_PER_SUB)],
                    out_hbm.at[c, pl.ds(s * ROWS_PER_SUB, ROWS_PER_SUB)])


_hop = functools.partial(
    pl.kernel,
    _hop_body,
    out_type=jax.ShapeDtypeStruct((NC, NPAD, D), jnp.float32),
    mesh=_MESH,
    scratch_types=[
        pltpu.VMEM((NCHH, CH), jnp.int32),
        pltpu.VMEM((NCHH, CH), jnp.int32),
        pltpu.VMEM((CH, D), jnp.float32),
        pltpu.VMEM((CH, D), jnp.float32),
        pltpu.VMEM_SHARED((NPAD, D), jnp.float32),
        pltpu.SemaphoreType.DMA,
        pltpu.SemaphoreType.DMA,
    ],
)()


def _prep_body(dstp, batchp, ones_hbm, z_hbm, deg_out, cnt_out,
               didx, bidx, ones_v, dacc, cacc):
    c = lax.axis_index("c")
    s = lax.axis_index("s")
    w = c * NS + s
    pltpu.sync_copy(dstp.at[w], didx)
    pltpu.sync_copy(batchp.at[w], bidx)
    pltpu.sync_copy(ones_hbm, ones_v)
    pltpu.sync_copy(z_hbm, dacc.at[pl.ds(s * ROWS_PER_SUB, ROWS_PER_SUB)])

    @pl.when(s == 0)
    def _():
        pltpu.sync_copy(z_hbm.at[pl.ds(0, GPAD)], cacc)

    plsc.subcore_barrier()

    def body(j, carry):
        pltpu.sync_copy(ones_v, dacc.at[didx.at[j]], add=True)
        return carry

    lax.fori_loop(0, NCH, body, 0)

    def body2(k, carry):
        pltpu.sync_copy(ones_v.at[pl.ds(0, 64)], cacc.at[bidx.at[k]], add=True)
        return carry

    lax.fori_loop(0, ROWS_PER_TILE // 64, body2, 0)
    plsc.subcore_barrier()
    pltpu.sync_copy(dacc.at[pl.ds(s * ROWS_PER_SUB, ROWS_PER_SUB)],
                    deg_out.at[c, pl.ds(s * ROWS_PER_SUB, ROWS_PER_SUB)])
    pltpu.sync_copy(cacc.at[pl.ds(s * (GPAD // NS), GPAD // NS)],
                    cnt_out.at[c, pl.ds(s * (GPAD // NS), GPAD // NS)])


_prep = functools.partial(
    pl.kernel,
    _prep_body,
    out_type=(jax.ShapeDtypeStruct((NC, NPAD, D), jnp.float32),
              jax.ShapeDtypeStruct((NC, GPAD, D), jnp.float32)),
    mesh=_MESH,
    scratch_types=[
        pltpu.VMEM((NCH, CH), jnp.int32),
        pltpu.VMEM((ROWS_PER_TILE // 64, 64), jnp.int32),
        pltpu.VMEM((CH, D), jnp.float32),
        pltpu.VMEM_SHARED((NPAD, D), jnp.float32),
        pltpu.VMEM_SHARED((GPAD, D), jnp.float32),
    ],
)()


def _pool_body(h_hbm, batchp, z_hbm, out_hbm, bidx, rows, pacc):
    c = lax.axis_index("c")
    s = lax.axis_index("s")
    w = c * NS + s
    pltpu.sync_copy(batchp.at[w], bidx)

    @pl.when(s == 0)
    def _():
        pltpu.sync_copy(z_hbm.at[pl.ds(0, GPAD)], pacc)

    plsc.subcore_barrier()

    def body(k, carry):
        pltpu.sync_copy(h_hbm.at[pl.ds(w * ROWS_PER_TILE + k * 64, 64)], rows)
        pltpu.sync_copy(rows, pacc.at[bidx.at[k]], add=True)
        return carry

    lax.fori_loop(0, ROWS_PER_TILE // 64, body, 0)
    plsc.subcore_barrier()
    pltpu.sync_copy(pacc.at[pl.ds(s * (GPAD // NS), GPAD // NS)],
                    out_hbm.at[c, pl.ds(s * (GPAD // NS), GPAD // NS)])


_pool = functools.partial(
    pl.kernel,
    _pool_body,
    out_type=jax.ShapeDtypeStruct((NC, GPAD, D), jnp.float32),
    mesh=_MESH,
    scratch_types=[
        pltpu.VMEM((ROWS_PER_TILE // 64, 64), jnp.int32),
        pltpu.VMEM((64, D), jnp.float32),
        pltpu.VMEM_SHARED((GPAD, D), jnp.float32),
    ],
)()


_BN = 1024
_NBLK = NPAD // _BN


def _dinv_body(da_ref, db_ref, out_ref):
    i = pl.program_id(0)
    deg = da_ref[:, :1] + db_ref[:, :1] + 1.0
    row = lax.broadcasted_iota(jnp.int32, (_BN, 1), 0) + i * _BN
    dv = jnp.where(row < N, 1.0 / jnp.sqrt(deg), 0.0)
    out_ref[...] = jnp.broadcast_to(dv, (_BN, D))


def _dinv_tc(da, db):
    return pl.pallas_call(
        _dinv_body,
        grid=(_NBLK,),
        in_specs=[pl.BlockSpec((_BN, D), lambda i: (i, 0)),
                  pl.BlockSpec((_BN, D), lambda i: (i, 0))],
        out_specs=pl.BlockSpec((_BN, D), lambda i: (i, 0)),
        out_shape=jax.ShapeDtypeStruct((NPAD, D), jnp.float32),
    )(da, db)


def _embed_body(x_ref, w_ref, b_ref, dv_ref, out_ref):
    h = jnp.dot(x_ref[...], w_ref[...], preferred_element_type=jnp.float32)
    out_ref[...] = dv_ref[...] * (h + b_ref[...])


def _embed_tc(x, w, b, dv):
    return pl.pallas_call(
        _embed_body,
        grid=(_NBLK,),
        in_specs=[pl.BlockSpec((_BN, D), lambda i: (i, 0)),
                  pl.BlockSpec((D, D), lambda i: (0, 0)),
                  pl.BlockSpec((1, D), lambda i: (0, 0)),
                  pl.BlockSpec((_BN, D), lambda i: (i, 0))],
        out_specs=pl.BlockSpec((_BN, D), lambda i: (i, 0)),
        out_shape=jax.ShapeDtypeStruct((NPAD, D), jnp.float32),
    )(x, w, b, dv)


def _comb2_body(a_ref, b_ref, g_ref, dv_ref, out_ref):
    dv = dv_ref[...]
    out_ref[...] = dv * dv * (a_ref[...] + b_ref[...] + g_ref[...])


def _comb2_tc(a, b, g, dv):
    return pl.pallas_call(
        _comb2_body,
        grid=(_NBLK,),
        in_specs=[pl.BlockSpec((_BN, D), lambda i: (i, 0))] * 4,
        out_specs=pl.BlockSpec((_BN, D), lambda i: (i, 0)),
        out_shape=jax.ShapeDtypeStruct((NPAD, D), jnp.float32),
    )(a, b, g, dv)


def _comb_mm_body(relu, post_scale, a_ref, b_ref, g_ref, dv_ref, w_ref,
                  bias_ref, out_ref):
    dv = dv_ref[...]
    t = dv * (a_ref[...] + b_ref[...] + g_ref[...])
    r = jnp.dot(t, w_ref[...], preferred_element_type=jnp.float32) + bias_ref[...]
    if relu:
        r = jnp.maximum(r, 0.0)
    if post_scale:
        r = dv * r
    out_ref[...] = r


def _comb_mm_tc(a, b, g, dv, w, bias, relu, post_scale):
    return pl.pallas_call(
        functools.partial(_comb_mm_body, relu, post_scale),
        grid=(_NBLK,),
        in_specs=[pl.BlockSpec((_BN, D), lambda i: (i, 0)),
                  pl.BlockSpec((_BN, D), lambda i: (i, 0)),
                  pl.BlockSpec((_BN, D), lambda i: (i, 0)),
                  pl.BlockSpec((_BN, D), lambda i: (i, 0)),
                  pl.BlockSpec((D, D), lambda i: (0, 0)),
                  pl.BlockSpec((1, D), lambda i: (0, 0))],
        out_specs=pl.BlockSpec((_BN, D), lambda i: (i, 0)),
        out_shape=jax.ShapeDtypeStruct((NPAD, D), jnp.float32),
    )(a, b, g, dv, w, bias)


def _task_body(pa_ref, pb_ref, ca_ref, cb_ref, w1_ref, b1_ref, gm_ref,
               bt_ref, w2_ref, b2_ref, out_ref):
    cnt = ca_ref[:, :1] + cb_ref[:, :1]
    pooled = (pa_ref[...] + pb_ref[...]) / jnp.maximum(cnt, 1.0)
    z = jnp.dot(pooled, w1_ref[...], preferred_element_type=jnp.float32) + b1_ref[...]
    mu = jnp.mean(z, axis=0, keepdims=True)
    var = jnp.mean((z - mu) ** 2, axis=0, keepdims=True)
    z = (z - mu) / jnp.sqrt(var + 1e-5) * gm_ref[...] + bt_ref[...]
    z = jnp.maximum(z, 0.0)
    out_ref[...] = jnp.dot(z, w2_ref[...], preferred_element_type=jnp.float32,
                           precision=lax.Precision.HIGHEST) + b2_ref[...]


def _task_tc(pa, pb, ca, cb, w1, b1, gm, bt, w2, b2):
    return pl.pallas_call(
        _task_body,
        out_shape=jax.ShapeDtypeStruct((G, D), jnp.float32),
    )(pa, pb, ca, cb, w1, b1, gm, bt, w2, b2)


def kernel(x, edge_index, batch, W_embed, b_embed, W1, b1, W2, b2,
           Wt1, bt1, gamma, beta, Wt2, bt2):
    f32 = jnp.float32
    i32 = jnp.int32

    xp = jnp.pad(x, ((0, NPAD - N), (0, 0)))
    src = edge_index[0].astype(i32)
    dst = edge_index[1].astype(i32)
    epad = jnp.full((NW * EC - E,), DUMMY, i32)
    srcp = jnp.concatenate([src, epad]).reshape(NW, NCH, CH)
    dstp = jnp.concatenate([dst, epad]).reshape(NW, NCH, CH)
    batchp = jnp.concatenate(
        [batch.astype(i32), jnp.full((NPAD - N,), G, i32)]
    ).reshape(NW, ROWS_PER_TILE // 64, 64)

    z640 = jnp.zeros((ROWS_PER_SUB, D), f32)
    ones_rows = jnp.ones((CH, D), f32)

    b_embed2 = b_embed.reshape(1, D)
    b1_2 = b1.reshape(1, D)
    b2_2 = b2.reshape(1, D)
    bt1_2 = bt1.reshape(1, D)
    gamma2 = gamma.reshape(1, D)
    beta2 = beta.reshape(1, D)
    Wt2p = jnp.pad(Wt2, ((0, 0), (0, D - Wt2.shape[1])))
    bt2p = jnp.pad(bt2, (0, D - bt2.shape[0])).reshape(1, D)

    deg2, cnt2 = _prep(dstp, batchp, ones_rows, z640)
    dinv = _dinv_tc(deg2[0], deg2[1])

    g0 = _embed_tc(xp, W_embed, b_embed2, dinv)
    o1 = _hop(g0, srcp, dstp, z640)
    g1 = _comb2_tc(o1[0], o1[1], g0, dinv)
    o2 = _hop(g1, srcp, dstp, z640)
    g3 = _comb_mm_tc(o2[0], o2[1], g1, dinv, W1, b1_2,
                     relu=True, post_scale=True)
    o3 = _hop(g3, srcp, dstp, z640)
    g4 = _comb2_tc(o3[0], o3[1], g3, dinv)
    o4 = _hop(g4, srcp, dstp, z640)
    h6 = _comb_mm_tc(o4[0], o4[1], g4, dinv, W2, b2_2,
                     relu=False, post_scale=False)

    po = _pool(h6, batchp, z640)
    zt = _task_tc(po[0, :G], po[1, :G], cnt2[0, :G], cnt2[1, :G],
                  Wt1, bt1_2, gamma2, beta2, Wt2p, bt2p)
    return zt[:, :1]
